# Initial kernel scaffold; baseline (speedup 1.0000x reference)
import jax
import jax.numpy as jnp
from jax.experimental import pallas as pl


def kernel(embeddings, ref, adj, W, W_loop, b):
    raise NotImplementedError("write your pallas kernel here")


# speedup vs baseline: 3.8814x; 3.8814x over previous
import jax
import jax.numpy as jnp
from jax.experimental import pallas as pl


def _mdm_kernel(feat_ref, refp_ref, adj_ref, w_ref, wl_ref, bias_ref, out_ref):
    feats = feat_ref[...]
    refp = refp_ref[...]
    w = w_ref[...]
    wl = wl_ref[...]
    bias = bias_ref[...]

    nf = feats.shape[1]
    wf, wr = w[:nf, :], w[nf:, :]
    wlf, wlr = wl[:nf, :], wl[nf:, :]

    s = jnp.dot(feats, wf, preferred_element_type=jnp.float32)
    sl = jnp.dot(feats, wlf, preferred_element_type=jnp.float32)
    r = jnp.dot(refp, wr, preferred_element_type=jnp.float32)
    rl = jnp.dot(refp, wlr, preferred_element_type=jnp.float32)

    raug = jnp.concatenate(
        [r, jnp.ones((refp.shape[0], 1), jnp.float32)], axis=1)
    ar = jnp.dot(adj_ref[...], raug, preferred_element_type=jnp.float32)
    neigh, rowsum = ar[:, :3], ar[:, 3:4]

    per_point = neigh + rl + bias
    out = (per_point[None, :, :]
           + rowsum[None, :, :] * s[:, None, :]
           + sl[:, None, :])
    out_ref[...] = jnp.tanh(out)


def kernel(embeddings, ref, adj, W, W_loop, b):
    n, batch, f_dim = embeddings.shape
    point_num = ref.shape[0]
    feats = jnp.transpose(embeddings, (1, 0, 2)).reshape(batch, n * f_dim)
    return pl.pallas_call(
        _mdm_kernel,
        out_shape=jax.ShapeDtypeStruct((batch, point_num, 3), jnp.float32),
    )(feats, ref, adj, W, W_loop, b.reshape(1, 3))


# speedup vs baseline: 5.6628x; 1.4589x over previous
import jax
import jax.numpy as jnp
from jax.experimental import pallas as pl


def _mdm_kernel(emb_ref, refp_ref, adj_ref, wc_ref, bias_ref, out_ref):
    n, batch, f_dim = emb_ref.shape
    nf = n * f_dim
    P = adj_ref.shape[0]
    wc = wc_ref[...]
    refp = refp_ref[...]

    ssl = jnp.zeros((batch, 6), jnp.float32)
    for i in range(n):
        ssl = ssl + jnp.dot(emb_ref[i], wc[i * f_dim:(i + 1) * f_dim, :],
                            preferred_element_type=jnp.float32)
    s, sl = ssl[:, :3], ssl[:, 3:]

    dn = (((0,), (1,)), ((), ()))
    rrl_t = jax.lax.dot_general(wc[nf:, :], refp, dn,
                                preferred_element_type=jnp.float32)
    raug_t = jnp.concatenate(
        [rrl_t[:3, :], jnp.ones((1, P), jnp.float32)], axis=0)

    dn2 = (((1,), (1,)), ((), ()))
    ar_t = jax.lax.dot_general(raug_t, adj_ref[...], dn2,
                               preferred_element_type=jnp.float32)
    neigh_t, rowsum_t = ar_t[:3, :], ar_t[3:4, :]

    per_point = neigh_t + rrl_t[3:, :] + bias_ref[...]
    out_t = (per_point[None]
             + rowsum_t[None] * s[:, :, None]
             + sl[:, :, None])
    out_ref[...] = jnp.tanh(out_t)


def kernel(embeddings, ref, adj, W, W_loop, b):
    n, batch, f_dim = embeddings.shape
    P = ref.shape[0]
    wc = jnp.concatenate([W, W_loop], axis=1)
    out_t = pl.pallas_call(
        _mdm_kernel,
        out_shape=jax.ShapeDtypeStruct((batch, 3, P), jnp.float32),
    )(embeddings, ref, adj, wc, b.reshape(3, 1))
    return jnp.swapaxes(out_t, 1, 2)
